# baseline (speedup 1.0000x reference)
import functools

import jax
import jax.numpy as jnp
from jax import lax
from jax.experimental import pallas as pl
from jax.experimental.pallas import tpu as pltpu
from jax.experimental.pallas import tpu_sc as plsc

NUM_CORES = 2
NUM_SUBCORES = 16
NUM_WORKERS = NUM_CORES * NUM_SUBCORES

SEQ_LEN = 10
EMBED_DIM = 256
TOTAL_ROWS = 16384 * SEQ_LEN

ROWS_PER_WORKER = TOTAL_ROWS // NUM_WORKERS
CHUNK = 80
CHUNKS_PER_WORKER = ROWS_PER_WORKER // CHUNK
REPS = CHUNK // SEQ_LEN
LANES = 16
VECS = EMBED_DIM // LANES


def _embed_body(idx_hbm, table_hbm, pos_hbm, out_hbm,
                idx_v, pos_v, rows_v, gsem0, gsem1, osem0, osem1):
    gsems = (gsem0, gsem1)
    osems = (osem0, osem1)
    wid = lax.axis_index("s") * NUM_CORES + lax.axis_index("c")
    base_row = wid * ROWS_PER_WORKER

    pltpu.sync_copy(idx_hbm.at[wid], idx_v)
    pltpu.sync_copy(pos_hbm, pos_v)

    def gather_start(g, buf):
        pltpu.async_copy(table_hbm.at[idx_v.at[g]], rows_v.at[buf],
                         gsems[buf])

    def gather_wait(g, buf):
        pltpu.make_async_copy(table_hbm.at[idx_v.at[g]], rows_v.at[buf],
                              gsems[buf]).wait()

    def out_start(g, buf):
        pltpu.async_copy(
            rows_v.at[buf], out_hbm.at[pl.ds(base_row + g * CHUNK, CHUNK)],
            osems[buf])

    def out_wait(g, buf):
        pltpu.make_async_copy(
            rows_v.at[buf], out_hbm.at[pl.ds(base_row + g * CHUNK, CHUNK)],
            osems[buf]).wait()

    def add_pos(buf):
        def rep_body(rep, carry):
            for l in range(SEQ_LEN):
                row = rep * SEQ_LEN + l
                for j in range(VECS):
                    sl = pl.ds(j * LANES, LANES)
                    rows_v[buf, row, sl] = rows_v[buf, row, sl] + pos_v[l, sl]
            return carry
        lax.fori_loop(0, REPS, rep_body, 0)

    gather_start(0, 0)
    gather_start(1, 1)

    def pair_body(p, carry):
        for b in range(2):
            g = p * 2 + b
            gather_wait(g, b)
            add_pos(b)
            out_start(g, b)
            @pl.when(g + 2 < CHUNKS_PER_WORKER)
            def _():
                out_wait(g, b)
                gather_start(g + 2, b)
        return carry

    lax.fori_loop(0, CHUNKS_PER_WORKER // 2, pair_body, 0)

    out_wait(CHUNKS_PER_WORKER - 2, 0)
    out_wait(CHUNKS_PER_WORKER - 1, 1)


def kernel(x, token_table, position_table):
    batch, seq_len = x.shape
    idx = x.reshape(NUM_WORKERS, CHUNKS_PER_WORKER, CHUNK).astype(jnp.int32)

    mesh = plsc.VectorSubcoreMesh(core_axis_name="c", subcore_axis_name="s")
    run = pl.kernel(
        _embed_body,
        out_type=jax.ShapeDtypeStruct((TOTAL_ROWS, EMBED_DIM), jnp.float32),
        mesh=mesh,
        scratch_types=[
            pltpu.VMEM((CHUNKS_PER_WORKER, CHUNK), jnp.int32),
            pltpu.VMEM((SEQ_LEN, EMBED_DIM), jnp.float32),
            pltpu.VMEM((2, CHUNK, EMBED_DIM), jnp.float32),
            pltpu.SemaphoreType.DMA,
            pltpu.SemaphoreType.DMA,
            pltpu.SemaphoreType.DMA,
            pltpu.SemaphoreType.DMA,
        ],
    )
    out = run(idx, token_table, position_table)
    return out.reshape(batch, seq_len, EMBED_DIM)


# speedup vs baseline: 5.6449x; 5.6449x over previous
import jax
import jax.numpy as jnp
from jax import lax
from jax.experimental import pallas as pl
from jax.experimental.pallas import tpu as pltpu
from jax.experimental.pallas import tpu_sc as plsc

NUM_CORES = 2
NUM_SUBCORES = 16
NUM_WORKERS = NUM_CORES * NUM_SUBCORES

BATCH = 16384
SEQ_LEN = 10
EMBED_DIM = 256
TOTAL_ROWS = BATCH * SEQ_LEN

ROWS_PER_WORKER = TOTAL_ROWS // NUM_WORKERS
CHUNK = 64
CHUNKS_PER_WORKER = ROWS_PER_WORKER // CHUNK
NBUF = 4
LANES = 16
VECS = EMBED_DIM // LANES
L_SHIFT = 14


def _embed_body(idx_hbm, table_hbm, pos_hbm, out_hbm, idx_v, pos_v, rows_v,
                gsem0, gsem1, gsem2, gsem3, osem0, osem1, osem2, osem3):
    gsems = (gsem0, gsem1, gsem2, gsem3)
    osems = (osem0, osem1, osem2, osem3)
    wid = lax.axis_index("s") * NUM_CORES + lax.axis_index("c")
    base_row = wid * ROWS_PER_WORKER

    pltpu.sync_copy(idx_hbm.at[wid], idx_v)
    pltpu.sync_copy(pos_hbm, pos_v)

    def gather_start(g, buf):
        pltpu.async_copy(table_hbm.at[idx_v.at[g]], rows_v.at[buf],
                         gsems[buf])

    def gather_wait(g, buf):
        pltpu.make_async_copy(table_hbm.at[idx_v.at[g]], rows_v.at[buf],
                              gsems[buf]).wait()

    def out_start(g, buf):
        pltpu.async_copy(
            rows_v.at[buf], out_hbm.at[pl.ds(base_row + g * CHUNK, CHUNK)],
            osems[buf])

    def out_wait(g, buf):
        pltpu.make_async_copy(
            rows_v.at[buf], out_hbm.at[pl.ds(base_row + g * CHUNK, CHUNK)],
            osems[buf]).wait()

    def add_pos(g, buf):
        l = (base_row + g * CHUNK) >> L_SHIFT
        prow = [pos_v[l, pl.ds(j * LANES, LANES)] for j in range(VECS)]

        def row_body(row, carry):
            for j in range(VECS):
                sl = pl.ds(j * LANES, LANES)
                rows_v[buf, row, sl] = rows_v[buf, row, sl] + prow[j]
            return carry
        lax.fori_loop(0, CHUNK, row_body, 0)

    gather_start(0, 0)
    gather_start(1, 1)

    def quad_body(p, carry):
        for b in range(NBUF):
            g = p * NBUF + b
            gather_wait(g, b)
            add_pos(g, b)
            out_start(g, b)
            @pl.when(g + 2 < CHUNKS_PER_WORKER)
            def _():
                @pl.when(g >= 2)
                def _():
                    out_wait(g - 2, (b + 2) % NBUF)
                gather_start(g + 2, (b + 2) % NBUF)
        return carry

    lax.fori_loop(0, CHUNKS_PER_WORKER // NBUF, quad_body, 0)

    for g in range(CHUNKS_PER_WORKER - NBUF, CHUNKS_PER_WORKER):
        out_wait(g, g % NBUF)


def kernel(x, token_table, position_table):
    batch, seq_len = x.shape
    idx = x.T.reshape(NUM_WORKERS, CHUNKS_PER_WORKER, CHUNK).astype(jnp.int32)

    mesh = plsc.VectorSubcoreMesh(core_axis_name="c", subcore_axis_name="s")
    run = pl.kernel(
        _embed_body,
        out_type=jax.ShapeDtypeStruct((TOTAL_ROWS, EMBED_DIM), jnp.float32),
        mesh=mesh,
        scratch_types=(
            [pltpu.VMEM((CHUNKS_PER_WORKER, CHUNK), jnp.int32),
             pltpu.VMEM((SEQ_LEN, EMBED_DIM), jnp.float32),
             pltpu.VMEM((NBUF, CHUNK, EMBED_DIM), jnp.float32)]
            + [pltpu.SemaphoreType.DMA] * (2 * NBUF)
        ),
    )
    out = run(idx, token_table, position_table)
    return out.reshape(seq_len, batch, EMBED_DIM).transpose(1, 0, 2)


# speedup vs baseline: 5.6821x; 1.0066x over previous
import jax
import jax.numpy as jnp
from jax import lax
from jax.experimental import pallas as pl
from jax.experimental.pallas import tpu as pltpu
from jax.experimental.pallas import tpu_sc as plsc

NUM_CORES = 2
NUM_SUBCORES = 16
NUM_WORKERS = NUM_CORES * NUM_SUBCORES

BATCH = 16384
SEQ_LEN = 10
EMBED_DIM = 256
TOTAL_ROWS = BATCH * SEQ_LEN

ROWS_PER_WORKER = TOTAL_ROWS // NUM_WORKERS
CHUNK = 128
CHUNKS_PER_WORKER = ROWS_PER_WORKER // CHUNK
NBUF = 3
LANES = 16
VECS = EMBED_DIM // LANES
L_SHIFT = 14


def _embed_body(idx_hbm, table_hbm, pos_hbm, out_hbm, idx_v, pos_v, rows_v,
                gsem0, gsem1, gsem2, osem0, osem1, osem2):
    gsems = (gsem0, gsem1, gsem2)
    osems = (osem0, osem1, osem2)
    wid = lax.axis_index("s") * NUM_CORES + lax.axis_index("c")
    base_row = wid * ROWS_PER_WORKER

    pltpu.sync_copy(idx_hbm.at[wid], idx_v)
    pltpu.sync_copy(pos_hbm, pos_v)

    def gather_start(g, buf):
        pltpu.async_copy(table_hbm.at[idx_v.at[g]], rows_v.at[buf],
                         gsems[buf])

    def gather_wait(g, buf):
        pltpu.make_async_copy(table_hbm.at[idx_v.at[g]], rows_v.at[buf],
                              gsems[buf]).wait()

    def out_start(g, buf):
        pltpu.async_copy(
            rows_v.at[buf], out_hbm.at[pl.ds(base_row + g * CHUNK, CHUNK)],
            osems[buf])

    def out_wait(g, buf):
        pltpu.make_async_copy(
            rows_v.at[buf], out_hbm.at[pl.ds(base_row + g * CHUNK, CHUNK)],
            osems[buf]).wait()

    def add_pos(g, buf):
        l = (base_row + g * CHUNK) >> L_SHIFT
        prow = [pos_v[l, pl.ds(j * LANES, LANES)] for j in range(VECS)]

        def row_body(row, carry):
            for j in range(VECS):
                sl = pl.ds(j * LANES, LANES)
                rows_v[buf, row, sl] = rows_v[buf, row, sl] + prow[j]
            return carry
        lax.fori_loop(0, CHUNK, row_body, 0)

    def step(g, b):
        gather_wait(g, b)
        add_pos(g, b)
        out_start(g, b)
        @pl.when(g + 2 < CHUNKS_PER_WORKER)
        def _():
            @pl.when(g >= 1)
            def _():
                out_wait(g - 1, (b + 2) % NBUF)
            gather_start(g + 2, (b + 2) % NBUF)

    gather_start(0, 0)
    gather_start(1, 1)

    def triple_body(p, carry):
        for b in range(NBUF):
            step(p * NBUF + b, b)
        return carry

    lax.fori_loop(0, (CHUNKS_PER_WORKER - 1) // NBUF, triple_body, 0)

    g_last = CHUNKS_PER_WORKER - 1
    gather_wait(g_last, g_last % NBUF)
    add_pos(g_last, g_last % NBUF)
    out_start(g_last, g_last % NBUF)

    for g in range(CHUNKS_PER_WORKER - NBUF, CHUNKS_PER_WORKER):
        out_wait(g, g % NBUF)


def kernel(x, token_table, position_table):
    batch, seq_len = x.shape
    idx = x.T.reshape(NUM_WORKERS, CHUNKS_PER_WORKER, CHUNK).astype(jnp.int32)

    mesh = plsc.VectorSubcoreMesh(core_axis_name="c", subcore_axis_name="s")
    run = pl.kernel(
        _embed_body,
        out_type=jax.ShapeDtypeStruct((TOTAL_ROWS, EMBED_DIM), jnp.float32),
        mesh=mesh,
        scratch_types=(
            [pltpu.VMEM((CHUNKS_PER_WORKER, CHUNK), jnp.int32),
             pltpu.VMEM((SEQ_LEN, EMBED_DIM), jnp.float32),
             pltpu.VMEM((NBUF, CHUNK, EMBED_DIM), jnp.float32)]
            + [pltpu.SemaphoreType.DMA] * (2 * NBUF)
        ),
    )
    out = run(idx, token_table, position_table)
    return out.reshape(seq_len, batch, EMBED_DIM).transpose(1, 0, 2)
